# baseline (speedup 1.0000x reference)
import functools

import jax
import jax.numpy as jnp
from jax import lax
from jax.experimental import pallas as pl
from jax.experimental.pallas import tpu as pltpu
from jax.experimental.pallas import tpu_sc as plsc

_N_S = 4096
_N_Q = 2048
_D = 512
_C = 2
_NEWTON_ITERS_BF16 = 6
_NEWTON_ITERS_WARM = 3
_NEWTON_ITERS_F32 = 2

_NW = 32
_RW = _N_S // _NW


_GDN = lax.GatherDimensionNumbers(offset_dims=(), collapsed_slice_dims=(0,),
                                  start_index_map=(0,))


def _splat(vec16, r):
    idx = jnp.full((16, 1), r, jnp.int32)
    return lax.gather(vec16, idx, _GDN, (1,),
                      mode=lax.GatherScatterMode.PROMISE_IN_BOUNDS)


def _sc_segsum_body(x_hbm, lab_hbm, sums_hbm, cnts_hbm, x_v, lab_v,
                    acc_v, m_v):
    cid = lax.axis_index("c")
    sid = lax.axis_index("s")
    wid = sid * 2 + cid
    base = wid * _RW
    pltpu.sync_copy(x_hbm.at[pl.ds(base, _RW)], x_v)
    pltpu.sync_copy(lab_hbm.at[pl.ds(base, _RW)], lab_v)

    zero16 = jnp.zeros((16,), jnp.float32)
    for r in range(_C):
        for cs in range(_D // 16):
            acc_v[r, pl.ds(cs * 16, 16)] = zero16

    def group(g, carry):
        lab16 = lab_v[pl.ds(g * 16, 16)]
        m16 = jnp.where(lab16 == 1, 1.0, 0.0).astype(jnp.float32)
        for r in range(16):
            row = x_v.at[g * 16 + r]
            ms = _splat(m16, r)
            for cs in range(_D // 16):
                xs = row[pl.ds(cs * 16, 16)]
                plsc.addupdate(acc_v.at[0, pl.ds(cs * 16, 16)], xs)
                plsc.addupdate(acc_v.at[1, pl.ds(cs * 16, 16)], xs * ms)
        return carry + m16

    cnt16 = lax.fori_loop(0, _RW // 16, group, zero16)
    for cs in range(_D // 16):
        sl = pl.ds(cs * 16, 16)
        acc_v[0, sl] = acc_v[0, sl] - acc_v[1, sl]
    pltpu.sync_copy(acc_v, sums_hbm.at[wid])

    m_v[...] = cnt16
    pltpu.sync_copy(m_v, cnts_hbm.at[wid])


def _sc_segment_sums(x, labels):
    mesh = plsc.VectorSubcoreMesh(core_axis_name="c", subcore_axis_name="s")
    fn = functools.partial(
        pl.kernel, mesh=mesh,
        out_type=[
            jax.ShapeDtypeStruct((_NW, _C, _D), jnp.float32),
            jax.ShapeDtypeStruct((_NW, 16), jnp.float32),
        ],
        scratch_types=[
            pltpu.VMEM((_RW, _D), jnp.float32),
            pltpu.VMEM((_RW,), jnp.int32),
            pltpu.VMEM((_C, _D), jnp.float32),
            pltpu.VMEM((16,), jnp.float32),
        ],
    )(_sc_segsum_body)
    return fn(x, labels)


def _grams_kernel(x_ref, lab_ref, g_ref):
    X = x_ref[...]
    labs = lab_ref[...]
    mask1 = (labs == 1).astype(jnp.bfloat16)

    dnums = (((0,), (0,)), ((), ()))
    X_hi = X.astype(jnp.bfloat16)
    Xm_hi = X_hi * mask1
    G_tot = jax.lax.dot_general(X_hi, X_hi, dnums,
                                preferred_element_type=jnp.float32)
    G_1 = jax.lax.dot_general(Xm_hi, Xm_hi, dnums,
                              preferred_element_type=jnp.float32)
    g_ref[...] = jnp.concatenate([G_tot, G_1], axis=0)


def _logits_kernel(g_ref, sums_ref, cnts_ref, q_ref, out_ref):
    G_tot = g_ref[:_D, :]
    G_1 = g_ref[_D:, :]
    G_0 = G_tot - G_1

    partials = sums_ref[...]
    s_0 = jnp.sum(partials[:, :_D], axis=0, keepdims=True)
    s_1 = jnp.sum(partials[:, _D:], axis=0, keepdims=True)
    s_tot = s_0 + s_1
    n1 = jnp.sum(cnts_ref[...])
    n0 = _N_S - n1

    m_all = s_tot / _N_S
    task_cov = (G_tot - _N_S * (m_all.T * m_all)) / (_N_S - 1.0)

    row = jax.lax.broadcasted_iota(jnp.int32, (_D, _D), 0)
    col = jax.lax.broadcasted_iota(jnp.int32, (_D, _D), 1)
    eye = (row == col).astype(jnp.float32)

    precisions = []
    means = []
    for c, (G_c, s_c, n_c) in enumerate(((G_0, s_0, n0), (G_1, s_1, n1))):
        m_c = s_c / n_c
        cov_c = (G_c - n_c * (m_c.T * m_c)) / (n_c - 1.0)
        lam = jnp.minimum(n_c / (n_c + 1.0), 0.1)
        S = lam * cov_c + (1.0 - lam) * task_cov + 0.1 * eye

        gersh = jnp.max(jnp.sum(jnp.abs(S), axis=1))
        c0 = 2.0 / (gersh + 0.1)

        S_bf = S.astype(jnp.bfloat16)

        def newton_bf16(_, P):
            SP = jnp.dot(S_bf, P, preferred_element_type=jnp.float32)
            T = (2.0 * eye - SP).astype(jnp.bfloat16)
            return jnp.dot(P, T,
                           preferred_element_type=jnp.float32
                           ).astype(jnp.bfloat16)

        if c == 0:
            P = jax.lax.fori_loop(0, _NEWTON_ITERS_BF16, newton_bf16,
                                  (c0 * eye).astype(jnp.bfloat16))
        else:
            P = jax.lax.fori_loop(0, _NEWTON_ITERS_WARM, newton_bf16,
                                  precisions[0].astype(jnp.bfloat16))
        P = P.astype(jnp.float32)

        def newton_polish(_, P):
            SP = jnp.dot(S, P, preferred_element_type=jnp.float32)
            E = (eye - SP).astype(jnp.bfloat16)
            dP = jnp.dot(P.astype(jnp.bfloat16), E,
                         preferred_element_type=jnp.float32)
            return P + dP

        P = jax.lax.fori_loop(0, _NEWTON_ITERS_F32, newton_polish, P)
        precisions.append(P)
        means.append(m_c)

    Q = q_ref[...]
    Q_bf = Q.astype(jnp.bfloat16)
    Pcat = jnp.concatenate(precisions, axis=1).astype(jnp.bfloat16)
    A = jnp.dot(Q_bf, Pcat, preferred_element_type=jnp.float32)

    logits = []
    for c in range(_C):
        A_c = A[:, c * _D:(c + 1) * _D]
        m_c = means[c]
        P_c = precisions[c]
        quad = jnp.sum(A_c * Q, axis=1, keepdims=True)
        cross = jnp.dot(A_c, m_c.T, preferred_element_type=jnp.float32)
        mP = jnp.dot(m_c, P_c, preferred_element_type=jnp.float32)
        const = jnp.sum(mP * m_c)
        logits.append(-(quad - 2.0 * cross + const))

    out_ref[...] = jnp.concatenate(logits, axis=1)


def kernel(support_features, query_features, support_labels):
    labels_i32 = support_labels.astype(jnp.int32)
    sums_part, cnts_part = _sc_segment_sums(support_features, labels_i32)
    sums_part = sums_part.reshape(_NW, _C * _D)

    labs2d = labels_i32.reshape(_N_S, 1)
    gcat = pl.pallas_call(
        _grams_kernel,
        out_shape=jax.ShapeDtypeStruct((2 * _D, _D), jnp.float32),
        compiler_params=pltpu.CompilerParams(
            vmem_limit_bytes=100 * 1024 * 1024,
        ),
    )(support_features, labs2d)

    return pl.pallas_call(
        _logits_kernel,
        out_shape=jax.ShapeDtypeStruct((_N_Q, _C), jnp.float32),
        compiler_params=pltpu.CompilerParams(
            vmem_limit_bytes=100 * 1024 * 1024,
        ),
    )(gcat, sums_part, cnts_part, query_features)


# speedup vs baseline: 2.4160x; 2.4160x over previous
import jax
import jax.numpy as jnp
from jax.experimental import pallas as pl
from jax.experimental.pallas import tpu as pltpu

_N_S = 4096
_N_Q = 2048
_D = 512
_C = 2
_NEWTON_ITERS_BF16 = 6
_NEWTON_ITERS_WARM = 3
_NEWTON_ITERS_F32 = 2


def _proto_kernel(x_ref, q_ref, lab_ref, out_ref):
    X = x_ref[...]
    labs = lab_ref[...]
    mask1 = (labs == 1).astype(jnp.float32)

    n1 = jnp.sum(mask1)
    n0 = _N_S - n1

    dnums = (((0,), (0,)), ((), ()))

    row = jax.lax.broadcasted_iota(jnp.int32, (_D, _D), 0)
    col = jax.lax.broadcasted_iota(jnp.int32, (_D, _D), 1)
    eye = (row == col).astype(jnp.float32)

    X_hi = X.astype(jnp.bfloat16)
    G_tot = jax.lax.dot_general(X_hi, X_hi, dnums,
                                preferred_element_type=jnp.float32)

    Xm_hi = X_hi * mask1.astype(jnp.bfloat16)
    G_1 = jax.lax.dot_general(Xm_hi, Xm_hi, dnums,
                              preferred_element_type=jnp.float32)
    G_0 = G_tot - G_1

    s_tot = jnp.sum(X, axis=0, keepdims=True)
    s_1 = jnp.sum(X * mask1, axis=0, keepdims=True)
    s_0 = s_tot - s_1

    m_all = s_tot / _N_S
    task_cov = (G_tot - _N_S * (m_all.T * m_all)) / (_N_S - 1.0)

    row = jax.lax.broadcasted_iota(jnp.int32, (_D, _D), 0)
    col = jax.lax.broadcasted_iota(jnp.int32, (_D, _D), 1)
    eye = (row == col).astype(jnp.float32)

    precisions = []
    means = []
    for c, (G_c, s_c, n_c) in enumerate(((G_0, s_0, n0), (G_1, s_1, n1))):
        m_c = s_c / n_c
        cov_c = (G_c - n_c * (m_c.T * m_c)) / (n_c - 1.0)
        lam = jnp.minimum(n_c / (n_c + 1.0), 0.1)
        S = lam * cov_c + (1.0 - lam) * task_cov + 0.1 * eye

        gersh = jnp.max(jnp.sum(jnp.abs(S), axis=1))
        c0 = 2.0 / (gersh + 0.1)

        S_bf = S.astype(jnp.bfloat16)

        def newton_bf16(_, P):
            SP = jnp.dot(S_bf, P, preferred_element_type=jnp.float32)
            T = (2.0 * eye - SP).astype(jnp.bfloat16)
            return jnp.dot(P, T,
                           preferred_element_type=jnp.float32
                           ).astype(jnp.bfloat16)

        if c == 0:
            P = jax.lax.fori_loop(0, _NEWTON_ITERS_BF16, newton_bf16,
                                  (c0 * eye).astype(jnp.bfloat16))
        else:
            P = jax.lax.fori_loop(0, _NEWTON_ITERS_WARM, newton_bf16,
                                  precisions[0].astype(jnp.bfloat16))
        P = P.astype(jnp.float32)

        def newton_polish(_, P):
            SP = jnp.dot(S, P, preferred_element_type=jnp.float32)
            E = (eye - SP).astype(jnp.bfloat16)
            dP = jnp.dot(P.astype(jnp.bfloat16), E,
                         preferred_element_type=jnp.float32)
            return P + dP

        P = jax.lax.fori_loop(0, _NEWTON_ITERS_F32, newton_polish, P)
        precisions.append(P)
        means.append(m_c)

    Q = q_ref[...]
    Q_bf = Q.astype(jnp.bfloat16)
    Pcat = jnp.concatenate(precisions, axis=1).astype(jnp.bfloat16)
    A = jnp.dot(Q_bf, Pcat, preferred_element_type=jnp.float32)

    logits = []
    for c in range(_C):
        A_c = A[:, c * _D:(c + 1) * _D]
        m_c = means[c]
        P_c = precisions[c]
        quad = jnp.sum(A_c * Q, axis=1, keepdims=True)
        cross = jnp.dot(A_c, m_c.T, preferred_element_type=jnp.float32)
        mP = jnp.dot(m_c, P_c, preferred_element_type=jnp.float32)
        const = jnp.sum(mP * m_c)
        logits.append(-(quad - 2.0 * cross + const))

    out_ref[...] = jnp.concatenate(logits, axis=1)


def kernel(support_features, query_features, support_labels):
    labs2d = support_labels.reshape(_N_S, 1).astype(jnp.int32)
    return pl.pallas_call(
        _proto_kernel,
        out_shape=jax.ShapeDtypeStruct((_N_Q, _C), jnp.float32),
        compiler_params=pltpu.CompilerParams(
            vmem_limit_bytes=100 * 1024 * 1024,
        ),
    )(support_features, query_features, labs2d)
